# baseline (speedup 1.0000x reference)
import functools

import jax
import jax.numpy as jnp
import numpy as np
from jax import lax
from jax.experimental import pallas as pl
from jax.experimental.pallas import tpu as pltpu
from jax.experimental.pallas import tpu_sc as plsc

_TABLE_SIZES = [999983, 999979, 999961, 999959]
_EMBED_DIM = 16
_NUM_HEADS = 4
_OFFSETS = np.concatenate([[0], np.cumsum(_TABLE_SIZES[:-1])]).astype(np.int32)

_INFO = plsc.get_sparse_core_info()
_NC = _INFO.num_cores
_NS = _INFO.num_subcores
_NW = _NC * _NS
_L = _INFO.num_lanes


def _make_sc_gather(n_rows: int, chunk: int):
    assert n_rows % _NW == 0
    per_w = n_rows // _NW
    assert per_w % chunk == 0
    n_chunks = per_w // chunk
    assert chunk % _L == 0

    mesh = plsc.VectorSubcoreMesh(core_axis_name="c", subcore_axis_name="s")

    @functools.partial(
        pl.kernel,
        mesh=mesh,
        out_type=jax.ShapeDtypeStruct((n_rows, _EMBED_DIM), jnp.float32),
        scratch_types=[
            pltpu.VMEM((chunk,), jnp.int32),
            pltpu.VMEM((chunk, _EMBED_DIM), jnp.float32),
            pltpu.VMEM((_L,), jnp.int32),
            pltpu.SemaphoreType.DMA,
        ],
        compiler_params=pltpu.CompilerParams(use_tc_tiling_on_sc=False),
    )
    def sc_kernel(ids_hbm, off_hbm, w_hbm, out_hbm, idx_v, rows_v, off_v, sem):
        wid = lax.axis_index("s") * _NC + lax.axis_index("c")
        base = wid * per_w
        pltpu.sync_copy(off_hbm, off_v)
        off = off_v[...]

        def chunk_body(c, _):
            cbase = base + c * chunk
            pltpu.sync_copy(ids_hbm.at[pl.ds(cbase, chunk)], idx_v)

            def add_body(i, _):
                sl = pl.ds(i * _L, _L)
                idx_v[sl] = idx_v[sl] + off
                return 0

            lax.fori_loop(0, chunk // _L, add_body, 0, unroll=8)
            pltpu.async_copy(w_hbm.at[idx_v], rows_v, sem).wait()
            pltpu.sync_copy(rows_v, out_hbm.at[pl.ds(cbase, chunk)])
            return 0

        lax.fori_loop(0, n_chunks, chunk_body, 0)

    return sc_kernel


@jax.jit
def kernel(hash_ids, weight):
    B, T, H = hash_ids.shape
    n_rows = B * T * H
    ids_flat = hash_ids.reshape(n_rows)
    off_tile = jnp.asarray(np.tile(_OFFSETS, _L // _NUM_HEADS), dtype=jnp.int32)
    out = _make_sc_gather(n_rows, 3200)(ids_flat, off_tile, weight)
    return out.reshape(B, T, H * _EMBED_DIM)
